# baseline (speedup 1.0000x reference)
import functools

import jax
import jax.numpy as jnp
from jax import lax
from jax.experimental import pallas as pl
from jax.experimental.pallas import tpu as pltpu
from jax.experimental.pallas import tpu_sc as plsc

B, H, W, C = 4, 384, 384, 96
N = B * H * W

NUM_CORES = 2
NUM_SUBCORES = 16
NUM_TILES = NUM_CORES * NUM_SUBCORES
PER_TILE = N // NUM_TILES
P = 128
CHUNKS = PER_TILE // P
CT, CI, XT = C // 8, 8, W // P


def _derive(base, fy_v, fx_v, i00_v, i01_v, i10_v, i11_v,
            w00_v, w01_v, w10_v, w11_v):
    lane = lax.broadcasted_iota(jnp.int32, (16,), 0)
    for g in range(P // 16):
        s = pl.ds(g * 16, 16)
        pix = base + g * 16 + lane
        row = pix // W
        j = pix - row * W
        i = lax.rem(row, H)
        qy = i.astype(jnp.float32) - fy_v[s]
        qx = j.astype(jnp.float32) - fx_v[s]
        qyc = jnp.clip(qy, 0.0, float(H - 2))
        qxc = jnp.clip(qx, 0.0, float(W - 2))
        y0 = qyc.astype(jnp.int32)
        x0 = qxc.astype(jnp.int32)
        ay = jnp.clip(qy - y0.astype(jnp.float32), 0.0, 1.0)
        ax = jnp.clip(qx - x0.astype(jnp.float32), 0.0, 1.0)
        idx = (row - i + y0) * W + x0
        i00_v[s] = idx
        i01_v[s] = idx + 1
        i10_v[s] = idx + W
        i11_v[s] = idx + W + 1
        by = 1.0 - ay
        bx = 1.0 - ax
        w00_v[s] = by * bx
        w01_v[s] = by * ax
        w10_v[s] = ay * bx
        w11_v[s] = ay * ax


def _blend(r00_v, r01_v, r10_v, r11_v, w00_v, w01_v, w10_v, w11_v, out_v):
    lane = lax.broadcasted_iota(jnp.int32, (16,), 0)

    def grp_body(g, cc):
        s = pl.ds(g * 16, 16)
        prow = g * 16 + lane
        w00 = w00_v[s]
        w01 = w01_v[s]
        w10 = w10_v[s]
        w11 = w11_v[s]

        def ct_body(ct, c2):
            cbase = jnp.full((16,), ct * CI, jnp.int32)
            for ci in range(CI):
                ccol = cbase + ci
                g00 = plsc.load_gather(r00_v, [prow, ccol])
                g01 = plsc.load_gather(r01_v, [prow, ccol])
                g10 = plsc.load_gather(r10_v, [prow, ccol])
                g11 = plsc.load_gather(r11_v, [prow, ccol])
                out_v[ct, ci, s] = (w00 * g00 + w01 * g01
                                    + w10 * g10 + w11 * g11)
            return c2

        lax.fori_loop(0, CT, ct_body, 0)
        return cc

    lax.fori_loop(0, P // 16, grp_body, 0)


def _warp_sc_body(img_hbm, fy_hbm, fx_hbm, out_hbm, *refs):
    (fy0, fy1, fx0, fx1,
     i00a, i01a, i10a, i11a, i00b, i01b, i10b, i11b,
     w00a, w01a, w10a, w11a, w00b, w01b, w10b, w11b,
     r00a, r01a, r10a, r11a, r00b, r01b, r10b, r11b,
     outa, outb,
     sem_f0, sem_f1, sem_g0, sem_g1, sem_o0, sem_o1) = refs

    wid = lax.axis_index("s") * NUM_CORES + lax.axis_index("c")
    tile_base = wid * PER_TILE

    bufs = (
        ((fy0, fx0), (i00a, i01a, i10a, i11a), (w00a, w01a, w10a, w11a),
         (r00a, r01a, r10a, r11a), outa, sem_f0, sem_g0, sem_o0),
        ((fy1, fx1), (i00b, i01b, i10b, i11b), (w00b, w01b, w10b, w11b),
         (r00b, r01b, r10b, r11b), outb, sem_f1, sem_g1, sem_o1),
    )

    def issue_flow(k, bi):
        fy_v, fx_v = bufs[bi][0]
        base = tile_base + k * P
        pltpu.async_copy(fy_hbm.at[pl.ds(base, P)], fy_v, bufs[bi][5])
        pltpu.async_copy(fx_hbm.at[pl.ds(base, P)], fx_v, bufs[bi][5])

    def wait_flow(bi):
        fy_v, fx_v = bufs[bi][0]
        pltpu.make_async_copy(fy_hbm.at[pl.ds(0, P)], fy_v, bufs[bi][5]).wait()
        pltpu.make_async_copy(fx_hbm.at[pl.ds(0, P)], fx_v, bufs[bi][5]).wait()

    def derive_issue_gathers(k, bi):
        (fy_v, fx_v), iv, wv, rv, _, _, sem_g, _ = bufs[bi]
        _derive(tile_base + k * P, fy_v, fx_v, *iv, *wv)
        pltpu.async_copy(img_hbm.at[iv[0]], rv[0], sem_g)
        pltpu.async_copy(img_hbm.at[iv[1]], rv[1], sem_g)
        pltpu.async_copy(img_hbm.at[iv[2]], rv[2], sem_g)
        pltpu.async_copy(img_hbm.at[iv[3]], rv[3], sem_g)

    def wait_gathers(bi):
        iv, rv, sem_g = bufs[bi][1], bufs[bi][3], bufs[bi][6]
        for q in range(4):
            pltpu.make_async_copy(img_hbm.at[iv[q]], rv[q], sem_g).wait()

    def issue_out(k, bi):
        out_v, sem_o = bufs[bi][4], bufs[bi][7]
        base = tile_base + k * P
        by = base // W
        xt = (base - by * W) // P
        pltpu.async_copy(out_v, out_hbm.at[by, :, xt], sem_o)

    def wait_out(bi):
        out_v, sem_o = bufs[bi][4], bufs[bi][7]
        pltpu.make_async_copy(out_v, out_hbm.at[0, :, 0], sem_o).wait()

    def blend(bi):
        _, _, wv, rv, out_v, _, _, _ = bufs[bi]
        _blend(*rv, *wv, out_v)

    issue_flow(0, 0)
    wait_flow(0)
    derive_issue_gathers(0, 0)
    issue_flow(1, 1)

    def pair_body(kk, cc):
        k = kk * 2
        wait_flow(1)
        derive_issue_gathers(k + 1, 1)

        @pl.when(kk + 1 < CHUNKS // 2)
        def _():
            issue_flow(k + 2, 0)

        wait_gathers(0)

        @pl.when(kk >= 1)
        def _():
            wait_out(0)

        blend(0)
        issue_out(k, 0)

        @pl.when(kk + 1 < CHUNKS // 2)
        def _():
            wait_flow(0)
            derive_issue_gathers(k + 2, 0)
            issue_flow(k + 3, 1)

        wait_gathers(1)

        @pl.when(kk >= 1)
        def _():
            wait_out(1)

        blend(1)
        issue_out(k + 1, 1)
        return cc

    lax.fori_loop(0, CHUNKS // 2, pair_body, 0)
    wait_out(0)
    wait_out(1)


@functools.cache
def _make_warp_sc():
    mesh = plsc.VectorSubcoreMesh(
        core_axis_name="c", subcore_axis_name="s",
        num_cores=NUM_CORES, num_subcores=NUM_SUBCORES,
    )
    idx_t = pltpu.VMEM((P,), jnp.int32)
    wgt_t = pltpu.VMEM((P,), jnp.float32)
    row_t = pltpu.VMEM((P, C), jnp.float32)
    out_t = pltpu.VMEM((CT, CI, P), jnp.float32)
    return pl.kernel(
        _warp_sc_body,
        out_type=jax.ShapeDtypeStruct((B * H, CT, XT, CI, P), jnp.float32),
        mesh=mesh,
        compiler_params=pltpu.CompilerParams(
            needs_layout_passes=False, use_tc_tiling_on_sc=False),
        scratch_types=(
            [wgt_t] * 4
            + [idx_t] * 8
            + [wgt_t] * 8
            + [row_t] * 8
            + [out_t] * 2
            + [pltpu.SemaphoreType.DMA] * 6
        ),
    )


def kernel(frame_tail, flow):
    fy = flow[..., 0].reshape(N)
    fx = flow[..., 1].reshape(N)
    warp = _make_warp_sc()
    out = warp(frame_tail.reshape(N, C), fy, fx)
    return (out.transpose(0, 2, 4, 1, 3)
               .reshape(B, H, W, C))


# speedup vs baseline: 3.3283x; 3.3283x over previous
import functools

import jax
import jax.numpy as jnp
from jax import lax
from jax.experimental import pallas as pl
from jax.experimental.pallas import tpu as pltpu
from jax.experimental.pallas import tpu_sc as plsc

B, H, W, C = 4, 384, 384, 96
N = B * H * W

NUM_CORES = 2
NUM_SUBCORES = 16
NUM_TILES = NUM_CORES * NUM_SUBCORES
PER_TILE = N // NUM_TILES
P = 128
CHUNKS = PER_TILE // P
CT, CI, XT = C // 8, 8, W // P


def _derive(base, fy_v, fx_v, i00_v, i01_v, i10_v, i11_v,
            w00_v, w01_v, w10_v, w11_v):
    lane = lax.broadcasted_iota(jnp.int32, (16,), 0)
    for g in range(P // 16):
        s = pl.ds(g * 16, 16)
        pix = base + g * 16 + lane
        row = pix // W
        j = pix - row * W
        i = lax.rem(row, H)
        qy = i.astype(jnp.float32) - fy_v[s]
        qx = j.astype(jnp.float32) - fx_v[s]
        qyc = jnp.clip(qy, 0.0, float(H - 2))
        qxc = jnp.clip(qx, 0.0, float(W - 2))
        y0 = qyc.astype(jnp.int32)
        x0 = qxc.astype(jnp.int32)
        ay = jnp.clip(qy - y0.astype(jnp.float32), 0.0, 1.0)
        ax = jnp.clip(qx - x0.astype(jnp.float32), 0.0, 1.0)
        idx = (row - i + y0) * W + x0
        i00_v[s] = idx
        i01_v[s] = idx + 1
        i10_v[s] = idx + W
        i11_v[s] = idx + W + 1
        by = 1.0 - ay
        bx = 1.0 - ax
        w00_v[s] = by * bx
        w01_v[s] = by * ax
        w10_v[s] = ay * bx
        w11_v[s] = ay * ax


def _blend(r00_v, r01_v, r10_v, r11_v, w00_v, w01_v, w10_v, w11_v, out_v):
    lane = lax.broadcasted_iota(jnp.int32, (16,), 0)
    ctv = []
    civ = []
    for u in range(C // 16):
        c = u * 16 + lane
        ctv.append(lax.shift_right_logical(c, 3))
        civ.append(lax.bitwise_and(c, 7))

    def px_body(p, cc):
        pcol = jnp.full((16,), p, jnp.int32)
        w00 = plsc.load_gather(w00_v, [pcol])
        w01 = plsc.load_gather(w01_v, [pcol])
        w10 = plsc.load_gather(w10_v, [pcol])
        w11 = plsc.load_gather(w11_v, [pcol])
        for u in range(C // 16):
            s = pl.ds(u * 16, 16)
            acc = (w00 * r00_v[p, s] + w01 * r01_v[p, s]
                   + w10 * r10_v[p, s] + w11 * r11_v[p, s])
            plsc.store_scatter(out_v, [ctv[u], civ[u], pcol], acc)
        return cc

    lax.fori_loop(0, P, px_body, 0, unroll=2)


def _warp_sc_body(img_hbm, fy_hbm, fx_hbm, out_hbm, *refs):
    (fy0, fy1, fx0, fx1,
     i00a, i01a, i10a, i11a, i00b, i01b, i10b, i11b,
     w00a, w01a, w10a, w11a, w00b, w01b, w10b, w11b,
     r00a, r01a, r10a, r11a, r00b, r01b, r10b, r11b,
     outa, outb,
     sem_f0, sem_f1, sem_g0, sem_g1, sem_o0, sem_o1) = refs

    wid = lax.axis_index("s") * NUM_CORES + lax.axis_index("c")
    tile_base = wid * PER_TILE

    bufs = (
        ((fy0, fx0), (i00a, i01a, i10a, i11a), (w00a, w01a, w10a, w11a),
         (r00a, r01a, r10a, r11a), outa, sem_f0, sem_g0, sem_o0),
        ((fy1, fx1), (i00b, i01b, i10b, i11b), (w00b, w01b, w10b, w11b),
         (r00b, r01b, r10b, r11b), outb, sem_f1, sem_g1, sem_o1),
    )

    def issue_flow(k, bi):
        fy_v, fx_v = bufs[bi][0]
        base = tile_base + k * P
        pltpu.async_copy(fy_hbm.at[pl.ds(base, P)], fy_v, bufs[bi][5])
        pltpu.async_copy(fx_hbm.at[pl.ds(base, P)], fx_v, bufs[bi][5])

    def wait_flow(bi):
        fy_v, fx_v = bufs[bi][0]
        pltpu.make_async_copy(fy_hbm.at[pl.ds(0, P)], fy_v, bufs[bi][5]).wait()
        pltpu.make_async_copy(fx_hbm.at[pl.ds(0, P)], fx_v, bufs[bi][5]).wait()

    def derive_issue_gathers(k, bi):
        (fy_v, fx_v), iv, wv, rv, _, _, sem_g, _ = bufs[bi]
        _derive(tile_base + k * P, fy_v, fx_v, *iv, *wv)
        pltpu.async_copy(img_hbm.at[iv[0]], rv[0], sem_g)
        pltpu.async_copy(img_hbm.at[iv[1]], rv[1], sem_g)
        pltpu.async_copy(img_hbm.at[iv[2]], rv[2], sem_g)
        pltpu.async_copy(img_hbm.at[iv[3]], rv[3], sem_g)

    def wait_gathers(bi):
        iv, rv, sem_g = bufs[bi][1], bufs[bi][3], bufs[bi][6]
        for q in range(4):
            pltpu.make_async_copy(img_hbm.at[iv[q]], rv[q], sem_g).wait()

    def issue_out(k, bi):
        out_v, sem_o = bufs[bi][4], bufs[bi][7]
        base = tile_base + k * P
        by = base // W
        xt = (base - by * W) // P
        pltpu.async_copy(out_v.at[:, :, pl.ds(0, P)], out_hbm.at[by, :, xt],
                         sem_o)

    def wait_out(bi):
        out_v, sem_o = bufs[bi][4], bufs[bi][7]
        pltpu.make_async_copy(out_v.at[:, :, pl.ds(0, P)],
                              out_hbm.at[0, :, 0], sem_o).wait()

    def blend(bi):
        _, _, wv, rv, out_v, _, _, _ = bufs[bi]
        _blend(*rv, *wv, out_v)

    issue_flow(0, 0)
    wait_flow(0)
    derive_issue_gathers(0, 0)
    issue_flow(1, 1)

    def pair_body(kk, cc):
        k = kk * 2
        wait_flow(1)
        derive_issue_gathers(k + 1, 1)

        @pl.when(kk + 1 < CHUNKS // 2)
        def _():
            issue_flow(k + 2, 0)

        wait_gathers(0)

        @pl.when(kk >= 1)
        def _():
            wait_out(0)

        blend(0)
        issue_out(k, 0)

        @pl.when(kk + 1 < CHUNKS // 2)
        def _():
            wait_flow(0)
            derive_issue_gathers(k + 2, 0)
            issue_flow(k + 3, 1)

        wait_gathers(1)

        @pl.when(kk >= 1)
        def _():
            wait_out(1)

        blend(1)
        issue_out(k + 1, 1)
        return cc

    lax.fori_loop(0, CHUNKS // 2, pair_body, 0)
    wait_out(0)
    wait_out(1)


@functools.cache
def _make_warp_sc():
    mesh = plsc.VectorSubcoreMesh(
        core_axis_name="c", subcore_axis_name="s",
        num_cores=NUM_CORES, num_subcores=NUM_SUBCORES,
    )
    idx_t = pltpu.VMEM((P,), jnp.int32)
    wgt_t = pltpu.VMEM((P,), jnp.float32)
    row_t = pltpu.VMEM((P, C), jnp.float32)
    out_t = pltpu.VMEM((CT, CI, P + 1), jnp.float32)
    return pl.kernel(
        _warp_sc_body,
        out_type=jax.ShapeDtypeStruct((B * H, CT, XT, CI, P), jnp.float32),
        mesh=mesh,
        compiler_params=pltpu.CompilerParams(
            needs_layout_passes=False, use_tc_tiling_on_sc=False),
        scratch_types=(
            [wgt_t] * 4
            + [idx_t] * 8
            + [wgt_t] * 8
            + [row_t] * 8
            + [out_t] * 2
            + [pltpu.SemaphoreType.DMA] * 6
        ),
    )


def kernel(frame_tail, flow):
    fy = flow[..., 0].reshape(N)
    fx = flow[..., 1].reshape(N)
    warp = _make_warp_sc()
    out = warp(frame_tail.reshape(N, C), fy, fx)
    return (out.transpose(0, 2, 4, 1, 3)
               .reshape(B, H, W, C))


# speedup vs baseline: 4.5077x; 1.3543x over previous
import functools

import jax
import jax.numpy as jnp
from jax import lax
from jax.experimental import pallas as pl
from jax.experimental.pallas import tpu as pltpu
from jax.experimental.pallas import tpu_sc as plsc

B, H, W, C = 4, 384, 384, 96
N = B * H * W

NUM_CORES = 2
NUM_SUBCORES = 16
NUM_TILES = NUM_CORES * NUM_SUBCORES
PER_TILE = N // NUM_TILES
P = 128
CHUNKS = PER_TILE // P
CT, CI, XT = C // 8, 8, W // P


def _derive(base, fy_v, fx_v, i00_v, i01_v, i10_v, i11_v,
            w00_v, w01_v, w10_v, w11_v):
    lane = lax.broadcasted_iota(jnp.int32, (16,), 0)
    for g in range(P // 16):
        s = pl.ds(g * 16, 16)
        pix = base + g * 16 + lane
        row = pix // W
        j = pix - row * W
        i = lax.rem(row, H)
        qy = i.astype(jnp.float32) - fy_v[s]
        qx = j.astype(jnp.float32) - fx_v[s]
        qyc = jnp.clip(qy, 0.0, float(H - 2))
        qxc = jnp.clip(qx, 0.0, float(W - 2))
        y0 = qyc.astype(jnp.int32)
        x0 = qxc.astype(jnp.int32)
        ay = jnp.clip(qy - y0.astype(jnp.float32), 0.0, 1.0)
        ax = jnp.clip(qx - x0.astype(jnp.float32), 0.0, 1.0)
        idx = (row - i + y0) * W + x0
        i00_v[s] = idx
        i01_v[s] = idx + 1
        i10_v[s] = idx + W
        i11_v[s] = idx + W + 1
        by = 1.0 - ay
        bx = 1.0 - ax
        w00_v[s] = by * bx
        w01_v[s] = by * ax
        w10_v[s] = ay * bx
        w11_v[s] = ay * ax


def _blend(r00_v, r01_v, r10_v, r11_v, w00_v, w01_v, w10_v, w11_v, out_v):
    lane = lax.broadcasted_iota(jnp.int32, (16,), 0)
    ctv = []
    civ = []
    for u in range(C // 16):
        c = u * 16 + lane
        ctv.append(lax.shift_right_logical(c, 3))
        civ.append(lax.bitwise_and(c, 7))

    def px_body(p, cc):
        pcol = jnp.full((16,), p, jnp.int32)
        w00 = plsc.load_gather(w00_v, [pcol])
        w01 = plsc.load_gather(w01_v, [pcol])
        w10 = plsc.load_gather(w10_v, [pcol])
        w11 = plsc.load_gather(w11_v, [pcol])
        NG = C // 16
        g00 = [r00_v[p, pl.ds(u * 16, 16)] for u in range(NG)]
        g01 = [r01_v[p, pl.ds(u * 16, 16)] for u in range(NG)]
        g10 = [r10_v[p, pl.ds(u * 16, 16)] for u in range(NG)]
        g11 = [r11_v[p, pl.ds(u * 16, 16)] for u in range(NG)]
        accs = [(w00 * g00[u] + w01 * g01[u])
                + (w10 * g10[u] + w11 * g11[u]) for u in range(NG)]
        for u in range(NG):
            plsc.store_scatter(out_v, [ctv[u], civ[u], pcol], accs[u])
        return cc

    lax.fori_loop(0, P, px_body, 0, unroll=2)


def _warp_sc_body(img_hbm, fy_hbm, fx_hbm, out_hbm, *refs):
    (fy0, fy1, fx0, fx1,
     i00a, i01a, i10a, i11a, i00b, i01b, i10b, i11b,
     w00a, w01a, w10a, w11a, w00b, w01b, w10b, w11b,
     r00a, r01a, r10a, r11a, r00b, r01b, r10b, r11b,
     outa, outb,
     sem_f0, sem_f1, sem_g0, sem_g1, sem_o0, sem_o1) = refs

    wid = lax.axis_index("s") * NUM_CORES + lax.axis_index("c")
    tile_base = wid * PER_TILE

    bufs = (
        ((fy0, fx0), (i00a, i01a, i10a, i11a), (w00a, w01a, w10a, w11a),
         (r00a, r01a, r10a, r11a), outa, sem_f0, sem_g0, sem_o0),
        ((fy1, fx1), (i00b, i01b, i10b, i11b), (w00b, w01b, w10b, w11b),
         (r00b, r01b, r10b, r11b), outb, sem_f1, sem_g1, sem_o1),
    )

    def issue_flow(k, bi):
        fy_v, fx_v = bufs[bi][0]
        base = tile_base + k * P
        pltpu.async_copy(fy_hbm.at[pl.ds(base, P)], fy_v, bufs[bi][5])
        pltpu.async_copy(fx_hbm.at[pl.ds(base, P)], fx_v, bufs[bi][5])

    def wait_flow(bi):
        fy_v, fx_v = bufs[bi][0]
        pltpu.make_async_copy(fy_hbm.at[pl.ds(0, P)], fy_v, bufs[bi][5]).wait()
        pltpu.make_async_copy(fx_hbm.at[pl.ds(0, P)], fx_v, bufs[bi][5]).wait()

    def derive_issue_gathers(k, bi):
        (fy_v, fx_v), iv, wv, rv, _, _, sem_g, _ = bufs[bi]
        _derive(tile_base + k * P, fy_v, fx_v, *iv, *wv)
        pltpu.async_copy(img_hbm.at[iv[0]], rv[0], sem_g)
        pltpu.async_copy(img_hbm.at[iv[1]], rv[1], sem_g)
        pltpu.async_copy(img_hbm.at[iv[2]], rv[2], sem_g)
        pltpu.async_copy(img_hbm.at[iv[3]], rv[3], sem_g)

    def wait_gathers(bi):
        iv, rv, sem_g = bufs[bi][1], bufs[bi][3], bufs[bi][6]
        for q in range(4):
            pltpu.make_async_copy(img_hbm.at[iv[q]], rv[q], sem_g).wait()

    def issue_out(k, bi):
        out_v, sem_o = bufs[bi][4], bufs[bi][7]
        base = tile_base + k * P
        by = base // W
        xt = (base - by * W) // P
        pltpu.async_copy(out_v.at[:, :, pl.ds(0, P)], out_hbm.at[by, :, xt],
                         sem_o)

    def wait_out(bi):
        out_v, sem_o = bufs[bi][4], bufs[bi][7]
        pltpu.make_async_copy(out_v.at[:, :, pl.ds(0, P)],
                              out_hbm.at[0, :, 0], sem_o).wait()

    def blend(bi):
        _, _, wv, rv, out_v, _, _, _ = bufs[bi]
        _blend(*rv, *wv, out_v)

    issue_flow(0, 0)
    wait_flow(0)
    derive_issue_gathers(0, 0)
    issue_flow(1, 1)

    def pair_body(kk, cc):
        k = kk * 2
        wait_flow(1)
        derive_issue_gathers(k + 1, 1)

        @pl.when(kk + 1 < CHUNKS // 2)
        def _():
            issue_flow(k + 2, 0)

        wait_gathers(0)

        @pl.when(kk >= 1)
        def _():
            wait_out(0)

        blend(0)
        issue_out(k, 0)

        @pl.when(kk + 1 < CHUNKS // 2)
        def _():
            wait_flow(0)
            derive_issue_gathers(k + 2, 0)
            issue_flow(k + 3, 1)

        wait_gathers(1)

        @pl.when(kk >= 1)
        def _():
            wait_out(1)

        blend(1)
        issue_out(k + 1, 1)
        return cc

    lax.fori_loop(0, CHUNKS // 2, pair_body, 0)
    wait_out(0)
    wait_out(1)


@functools.cache
def _make_warp_sc():
    mesh = plsc.VectorSubcoreMesh(
        core_axis_name="c", subcore_axis_name="s",
        num_cores=NUM_CORES, num_subcores=NUM_SUBCORES,
    )
    idx_t = pltpu.VMEM((P,), jnp.int32)
    wgt_t = pltpu.VMEM((P,), jnp.float32)
    row_t = pltpu.VMEM((P, C), jnp.float32)
    out_t = pltpu.VMEM((CT, CI, P + 1), jnp.float32)
    return pl.kernel(
        _warp_sc_body,
        out_type=jax.ShapeDtypeStruct((B * H, CT, XT, CI, P), jnp.float32),
        mesh=mesh,
        compiler_params=pltpu.CompilerParams(
            needs_layout_passes=False, use_tc_tiling_on_sc=False),
        scratch_types=(
            [wgt_t] * 4
            + [idx_t] * 8
            + [wgt_t] * 8
            + [row_t] * 8
            + [out_t] * 2
            + [pltpu.SemaphoreType.DMA] * 6
        ),
    )


def kernel(frame_tail, flow):
    fy = flow[..., 0].reshape(N)
    fx = flow[..., 1].reshape(N)
    warp = _make_warp_sc()
    out = warp(frame_tail.reshape(N, C), fy, fx)
    return (out.transpose(0, 2, 4, 1, 3)
               .reshape(B, H, W, C))


# speedup vs baseline: 4.5346x; 1.0060x over previous
import functools

import jax
import jax.numpy as jnp
from jax import lax
from jax.experimental import pallas as pl
from jax.experimental.pallas import tpu as pltpu
from jax.experimental.pallas import tpu_sc as plsc

B, H, W, C = 4, 384, 384, 96
N = B * H * W

NUM_CORES = 2
NUM_SUBCORES = 16
NUM_TILES = NUM_CORES * NUM_SUBCORES
PER_TILE = N // NUM_TILES
P = 128
CHUNKS = PER_TILE // P
CT, CI, XT = C // 8, 8, W // P


def _derive(base, fy_v, fx_v, i00_v, i01_v, i10_v, i11_v,
            w00_v, w01_v, w10_v, w11_v):
    lane = lax.broadcasted_iota(jnp.int32, (16,), 0)
    for g in range(P // 16):
        s = pl.ds(g * 16, 16)
        pix = base + g * 16 + lane
        row = pix // W
        j = pix - row * W
        i = lax.rem(row, H)
        qy = i.astype(jnp.float32) - fy_v[s]
        qx = j.astype(jnp.float32) - fx_v[s]
        qyc = jnp.clip(qy, 0.0, float(H - 2))
        qxc = jnp.clip(qx, 0.0, float(W - 2))
        y0 = qyc.astype(jnp.int32)
        x0 = qxc.astype(jnp.int32)
        ay = jnp.clip(qy - y0.astype(jnp.float32), 0.0, 1.0)
        ax = jnp.clip(qx - x0.astype(jnp.float32), 0.0, 1.0)
        idx = (row - i + y0) * W + x0
        i00_v[s] = idx
        i01_v[s] = idx + 1
        i10_v[s] = idx + W
        i11_v[s] = idx + W + 1
        by = 1.0 - ay
        bx = 1.0 - ax
        w00_v[s] = by * bx
        w01_v[s] = by * ax
        w10_v[s] = ay * bx
        w11_v[s] = ay * ax


def _blend(r00_v, r01_v, r10_v, r11_v, w00_v, w01_v, w10_v, w11_v, out_v):
    lane = lax.broadcasted_iota(jnp.int32, (16,), 0)
    ctv = []
    civ = []
    for u in range(C // 16):
        c = u * 16 + lane
        ctv.append(lax.shift_right_logical(c, 3))
        civ.append(lax.bitwise_and(c, 7))

    def px_body(p, cc):
        pcol = jnp.full((16,), p, jnp.int32)
        w00 = plsc.load_gather(w00_v, [pcol])
        w01 = plsc.load_gather(w01_v, [pcol])
        w10 = plsc.load_gather(w10_v, [pcol])
        w11 = plsc.load_gather(w11_v, [pcol])
        NG = C // 16
        g00 = [r00_v[p, pl.ds(u * 16, 16)] for u in range(NG)]
        g01 = [r01_v[p, pl.ds(u * 16, 16)] for u in range(NG)]
        g10 = [r10_v[p, pl.ds(u * 16, 16)] for u in range(NG)]
        g11 = [r11_v[p, pl.ds(u * 16, 16)] for u in range(NG)]
        accs = [(w00 * g00[u] + w01 * g01[u])
                + (w10 * g10[u] + w11 * g11[u]) for u in range(NG)]
        for u in range(NG):
            plsc.store_scatter(out_v, [ctv[u], civ[u], pcol], accs[u])
        return cc

    lax.fori_loop(0, P, px_body, 0, unroll=4)


def _warp_sc_body(img_hbm, fy_hbm, fx_hbm, out_hbm, *refs):
    (fy0, fy1, fx0, fx1,
     i00a, i01a, i10a, i11a, i00b, i01b, i10b, i11b,
     w00a, w01a, w10a, w11a, w00b, w01b, w10b, w11b,
     r00a, r01a, r10a, r11a, r00b, r01b, r10b, r11b,
     outa, outb,
     sem_f0, sem_f1, sem_g0, sem_g1, sem_o0, sem_o1) = refs

    wid = lax.axis_index("s") * NUM_CORES + lax.axis_index("c")
    tile_base = wid * PER_TILE

    bufs = (
        ((fy0, fx0), (i00a, i01a, i10a, i11a), (w00a, w01a, w10a, w11a),
         (r00a, r01a, r10a, r11a), outa, sem_f0, sem_g0, sem_o0),
        ((fy1, fx1), (i00b, i01b, i10b, i11b), (w00b, w01b, w10b, w11b),
         (r00b, r01b, r10b, r11b), outb, sem_f1, sem_g1, sem_o1),
    )

    def issue_flow(k, bi):
        fy_v, fx_v = bufs[bi][0]
        base = tile_base + k * P
        pltpu.async_copy(fy_hbm.at[pl.ds(base, P)], fy_v, bufs[bi][5])
        pltpu.async_copy(fx_hbm.at[pl.ds(base, P)], fx_v, bufs[bi][5])

    def wait_flow(bi):
        fy_v, fx_v = bufs[bi][0]
        pltpu.make_async_copy(fy_hbm.at[pl.ds(0, P)], fy_v, bufs[bi][5]).wait()
        pltpu.make_async_copy(fx_hbm.at[pl.ds(0, P)], fx_v, bufs[bi][5]).wait()

    def derive_issue_gathers(k, bi):
        (fy_v, fx_v), iv, wv, rv, _, _, sem_g, _ = bufs[bi]
        _derive(tile_base + k * P, fy_v, fx_v, *iv, *wv)
        pltpu.async_copy(img_hbm.at[iv[0]], rv[0], sem_g)
        pltpu.async_copy(img_hbm.at[iv[1]], rv[1], sem_g)
        pltpu.async_copy(img_hbm.at[iv[2]], rv[2], sem_g)
        pltpu.async_copy(img_hbm.at[iv[3]], rv[3], sem_g)

    def wait_gathers(bi):
        iv, rv, sem_g = bufs[bi][1], bufs[bi][3], bufs[bi][6]
        for q in range(4):
            pltpu.make_async_copy(img_hbm.at[iv[q]], rv[q], sem_g).wait()

    def issue_out(k, bi):
        out_v, sem_o = bufs[bi][4], bufs[bi][7]
        base = tile_base + k * P
        by = base // W
        xt = (base - by * W) // P
        pltpu.async_copy(out_v.at[:, :, pl.ds(0, P)], out_hbm.at[by, :, xt],
                         sem_o)

    def wait_out(bi):
        out_v, sem_o = bufs[bi][4], bufs[bi][7]
        pltpu.make_async_copy(out_v.at[:, :, pl.ds(0, P)],
                              out_hbm.at[0, :, 0], sem_o).wait()

    def blend(bi):
        _, _, wv, rv, out_v, _, _, _ = bufs[bi]
        _blend(*rv, *wv, out_v)

    issue_flow(0, 0)
    wait_flow(0)
    derive_issue_gathers(0, 0)
    issue_flow(1, 1)

    def pair_body(kk, cc):
        k = kk * 2
        wait_flow(1)
        derive_issue_gathers(k + 1, 1)

        @pl.when(kk + 1 < CHUNKS // 2)
        def _():
            issue_flow(k + 2, 0)

        wait_gathers(0)

        @pl.when(kk >= 1)
        def _():
            wait_out(0)

        blend(0)
        issue_out(k, 0)

        @pl.when(kk + 1 < CHUNKS // 2)
        def _():
            wait_flow(0)
            derive_issue_gathers(k + 2, 0)
            issue_flow(k + 3, 1)

        wait_gathers(1)

        @pl.when(kk >= 1)
        def _():
            wait_out(1)

        blend(1)
        issue_out(k + 1, 1)
        return cc

    lax.fori_loop(0, CHUNKS // 2, pair_body, 0)
    wait_out(0)
    wait_out(1)


@functools.cache
def _make_warp_sc():
    mesh = plsc.VectorSubcoreMesh(
        core_axis_name="c", subcore_axis_name="s",
        num_cores=NUM_CORES, num_subcores=NUM_SUBCORES,
    )
    idx_t = pltpu.VMEM((P,), jnp.int32)
    wgt_t = pltpu.VMEM((P,), jnp.float32)
    row_t = pltpu.VMEM((P, C), jnp.float32)
    out_t = pltpu.VMEM((CT, CI, P + 1), jnp.float32)
    return pl.kernel(
        _warp_sc_body,
        out_type=jax.ShapeDtypeStruct((B * H, CT, XT, CI, P), jnp.float32),
        mesh=mesh,
        compiler_params=pltpu.CompilerParams(
            needs_layout_passes=False, use_tc_tiling_on_sc=False),
        scratch_types=(
            [wgt_t] * 4
            + [idx_t] * 8
            + [wgt_t] * 8
            + [row_t] * 8
            + [out_t] * 2
            + [pltpu.SemaphoreType.DMA] * 6
        ),
    )


def kernel(frame_tail, flow):
    fy = flow[..., 0].reshape(N)
    fx = flow[..., 1].reshape(N)
    warp = _make_warp_sc()
    out = warp(frame_tail.reshape(N, C), fy, fx)
    return (out.transpose(0, 2, 4, 1, 3)
               .reshape(B, H, W, C))
